# baseline (speedup 1.0000x reference)
import jax
import jax.numpy as jnp
from jax import lax
from jax.experimental import pallas as pl


def _body(ids_ref, emb_ref, w_ref, b_ref, out_ref):
    tok = out_ref.shape[0]
    vocab = emb_ref.shape[0]
    ids = ids_ref[0]
    iota_v = lax.broadcasted_iota(jnp.int32, (vocab, tok), 0)
    ohT = jnp.where(iota_v == ids, 1.0, 0.0).astype(jnp.float32)
    hidden = lax.dot_general(
        ohT, emb_ref[...],
        dimension_numbers=(((0,), (0,)), ((), ())),
        preferred_element_type=jnp.float32,
    )
    logits = lax.dot_general(
        hidden, w_ref[...],
        dimension_numbers=(((1,), (1,)), ((), ())),
        preferred_element_type=jnp.float32,
    )
    out_ref[...] = logits + b_ref[...]


def kernel(input_ids, emb_table, lm_head_w, lm_head_b):
    B, L = input_ids.shape
    V, H = emb_table.shape
    N = B * L
    TOK = 512
    assert N % TOK == 0
    ids3 = input_ids.reshape(N // TOK, 1, TOK)
    bias2 = lm_head_b.reshape(1, V)

    out = pl.pallas_call(
        _body,
        grid=(N // TOK,),
        in_specs=[
            pl.BlockSpec((1, 1, TOK), lambda i: (i, 0, 0)),
            pl.BlockSpec((V, H), lambda i: (0, 0)),
            pl.BlockSpec((V, H), lambda i: (0, 0)),
            pl.BlockSpec((1, V), lambda i: (0, 0)),
        ],
        out_specs=pl.BlockSpec((TOK, V), lambda i: (i, 0)),
        out_shape=jax.ShapeDtypeStruct((N, V), jnp.float32),
    )(ids3, emb_table, lm_head_w, bias2)
    return out.reshape(B, L, V)


# speedup vs baseline: 1.2563x; 1.2563x over previous
import jax
import jax.numpy as jnp
from jax import lax
from jax.experimental import pallas as pl


def _body(ids_ref, emb_ref, w_ref, b_ref, out_ref):
    bb, ll = ids_ref.shape
    vocab = emb_ref.shape[0]
    ids3 = ids_ref[...][:, :, None]
    iota_v = lax.broadcasted_iota(jnp.int32, (bb, ll, vocab), 2)
    onehot = jnp.where(iota_v == ids3, 1.0, 0.0).astype(jnp.float32)
    hidden = lax.dot_general(
        onehot, emb_ref[...],
        dimension_numbers=(((2,), (0,)), ((), ())),
        preferred_element_type=jnp.float32,
    )
    logits = lax.dot_general(
        hidden, w_ref[...],
        dimension_numbers=(((2,), (1,)), ((), ())),
        preferred_element_type=jnp.float32,
    )
    out_ref[...] = logits + b_ref[...]


def kernel(input_ids, emb_table, lm_head_w, lm_head_b):
    B, L = input_ids.shape
    V, H = emb_table.shape
    BB = 8
    assert B % BB == 0
    bias3 = lm_head_b.reshape(1, 1, V)

    return pl.pallas_call(
        _body,
        grid=(B // BB,),
        in_specs=[
            pl.BlockSpec((BB, L), lambda i: (i, 0)),
            pl.BlockSpec((V, H), lambda i: (0, 0)),
            pl.BlockSpec((V, H), lambda i: (0, 0)),
            pl.BlockSpec((1, 1, V), lambda i: (0, 0, 0)),
        ],
        out_specs=pl.BlockSpec((BB, L, V), lambda i: (i, 0, 0)),
        out_shape=jax.ShapeDtypeStruct((B, L, V), jnp.float32),
    )(input_ids, emb_table, lm_head_w, bias3)


# speedup vs baseline: 1.3776x; 1.0966x over previous
import jax
import jax.numpy as jnp
from jax import lax
from jax.experimental import pallas as pl


def _body(ids_ref, emb_ref, w_ref, b_ref, out_ref):
    bb, ll = ids_ref.shape
    vocab = emb_ref.shape[0]
    ids3 = ids_ref[...][:, :, None]
    iota_v = lax.broadcasted_iota(jnp.int32, (bb, ll, vocab), 2)
    onehot = jnp.where(iota_v == ids3, 1.0, 0.0).astype(jnp.float32)
    hidden = lax.dot_general(
        onehot, emb_ref[...],
        dimension_numbers=(((2,), (0,)), ((), ())),
        preferred_element_type=jnp.float32,
    )
    logits = lax.dot_general(
        hidden, w_ref[...],
        dimension_numbers=(((2,), (1,)), ((), ())),
        preferred_element_type=jnp.float32,
    )
    out_ref[...] = logits + b_ref[...]


def kernel(input_ids, emb_table, lm_head_w, lm_head_b):
    B, L = input_ids.shape
    V, H = emb_table.shape
    BB = 32
    assert B % BB == 0
    bias3 = lm_head_b.reshape(1, 1, V)

    return pl.pallas_call(
        _body,
        grid=(B // BB,),
        in_specs=[
            pl.BlockSpec((BB, L), lambda i: (i, 0)),
            pl.BlockSpec((V, H), lambda i: (0, 0)),
            pl.BlockSpec((V, H), lambda i: (0, 0)),
            pl.BlockSpec((1, 1, V), lambda i: (0, 0, 0)),
        ],
        out_specs=pl.BlockSpec((BB, L, V), lambda i: (i, 0, 0)),
        out_shape=jax.ShapeDtypeStruct((B, L, V), jnp.float32),
    )(input_ids, emb_table, lm_head_w, bias3)
